# baseline (speedup 1.0000x reference)
import functools

import jax
import jax.numpy as jnp
from jax import lax
from jax.experimental import pallas as pl
from jax.experimental.pallas import tpu as pltpu
from jax.experimental.pallas import tpu_sc as plsc

_f32 = jnp.float32
_i32 = jnp.int32

N = 10000
D = 128
NC, NS = 2, 16
NW = NC * NS
CH = 128
K = 81
EPW = K * CH
EP = NW * EPW
NP = 10240
RW = NP // NS
ZR = 64


def _mesh():
    return plsc.VectorSubcoreMesh(core_axis_name="c", subcore_axis_name="s")


def _degree_call(colp, ewp):
    sds = jax.ShapeDtypeStruct((2, N), _f32)

    @functools.partial(
        pl.kernel,
        out_type=sds,
        mesh=_mesh(),
        scratch_types=[
            pltpu.VMEM((K, CH), _i32),
            pltpu.VMEM((EPW,), _f32),
            pltpu.VMEM((RW,), _f32),
            pltpu.VMEM_SHARED((NP,), _f32),
        ],
        compiler_params=pltpu.CompilerParams(use_tc_tiling_on_sc=False),
    )
    def deg_kernel(col_hbm, ew_hbm, deg_hbm, col_v, ew_v, z_v, deg_sh):
        cid = lax.axis_index("c")
        sid = lax.axis_index("s")
        wid = sid * NC + cid

        def zi(i, c):
            z_v[pl.ds(i * 16, 16)] = jnp.zeros((16,), _f32)
            return c

        lax.fori_loop(0, RW // 16, zi, 0)
        pltpu.sync_copy(z_v, deg_sh.at[pl.ds(sid * RW, RW)])
        pltpu.sync_copy(col_hbm.at[pl.ds(wid * K, K)], col_v)
        pltpu.sync_copy(ew_hbm.at[pl.ds(wid * EPW, EPW)], ew_v)
        plsc.subcore_barrier()

        def chunk(j, c):
            pltpu.sync_copy(ew_v.at[pl.ds(j * CH, CH)], deg_sh.at[col_v.at[j]],
                            add=True)
            return c

        lax.fori_loop(0, K, chunk, 0)
        plsc.subcore_barrier()
        base = jnp.minimum(sid * RW, N - RW)
        pltpu.sync_copy(deg_sh.at[pl.ds(base, RW)], z_v)

        @pl.when(cid == 0)
        def _():
            pltpu.sync_copy(z_v, deg_hbm.at[0, pl.ds(base, RW)])

        @pl.when(cid == 1)
        def _():
            pltpu.sync_copy(z_v, deg_hbm.at[1, pl.ds(base, RW)])

    return deg_kernel(colp, ewp)


DH = D // 2


def _transform_body(x_ref, w_ref, d_ref, g0_ref, g1_ref, dis_ref):
    h = lax.dot_general(
        x_ref[...], w_ref[...], (((1,), (1,)), ((), ())),
        preferred_element_type=_f32,
    )
    ones2 = jnp.ones((2, 1), _f32)
    dcol = lax.dot_general(d_ref[...], ones2, (((0,), (0,)), ((), ())),
                           precision=lax.Precision.HIGHEST,
                           preferred_element_type=_f32)
    r = lax.rsqrt(dcol + 1.0)
    g = h * r
    g0_ref[...] = g[:, :DH]
    g1_ref[...] = g[:, DH:]
    dis_ref[...] = r


def _transform_call(x, w, deg):
    return pl.pallas_call(
        _transform_body,
        grid=(1,),
        in_specs=[
            pl.BlockSpec((N, D), lambda i: (0, 0)),
            pl.BlockSpec((D, D), lambda i: (0, 0)),
            pl.BlockSpec((2, N), lambda i: (0, 0)),
        ],
        out_specs=[
            pl.BlockSpec((N, DH), lambda i: (0, 0)),
            pl.BlockSpec((N, DH), lambda i: (0, 0)),
            pl.BlockSpec((N, 1), lambda i: (0, 0)),
        ],
        out_shape=[
            jax.ShapeDtypeStruct((N, DH), _f32),
            jax.ShapeDtypeStruct((N, DH), _f32),
            jax.ShapeDtypeStruct((N, 1), _f32),
        ],
    )(x, w, deg)


def _propagate_call(g0, g1, rowp, colp, ewp):
    sds = jax.ShapeDtypeStruct((N, DH), _f32)

    NB = 3

    @functools.partial(
        pl.kernel,
        out_type=jax.ShapeDtypeStruct((4, N, DH), _f32),
        mesh=_mesh(),
        scratch_types=[
            pltpu.VMEM((K, CH), _i32),
            pltpu.VMEM((K, CH), _i32),
            pltpu.VMEM((EPW,), _f32),
            [pltpu.VMEM((CH, DH), _f32) for _ in range(NB)],
            [pltpu.VMEM((CH, DH), _f32) for _ in range(NB)],
            pltpu.VMEM((ZR, DH), _f32),
            pltpu.VMEM_SHARED((NP, DH), _f32),
            [pltpu.SemaphoreType.DMA for _ in range(NB)],
            [pltpu.SemaphoreType.DMA for _ in range(NB)],
        ],
        compiler_params=pltpu.CompilerParams(use_tc_tiling_on_sc=False),
    )
    def prop_kernel(g0_hbm, g1_hbm, row_hbm, col_hbm, ew_hbm, t_hbm,
                    row_v, col_v, ew_v, gbufs, sbufs, z_v, acc_sh,
                    gsems, ssems):
        cid = lax.axis_index("c")
        sid = lax.axis_index("s")
        wid = sid * NC + cid

        def zi(i, c):
            for u in range(DH // 16):
                z_v[i, pl.ds(u * 16, 16)] = jnp.zeros((16,), _f32)
            return c

        lax.fori_loop(0, ZR, zi, 0)
        pltpu.sync_copy(row_hbm.at[pl.ds(wid * K, K)], row_v)
        pltpu.sync_copy(col_hbm.at[pl.ds(wid * K, K)], col_v)
        pltpu.sync_copy(ew_hbm.at[pl.ds(wid * EPW, EPW)], ew_v)
        base = jnp.minimum(sid * RW, N - RW)
        NG = K // NB

        for h, g_hbm in ((0, g0_hbm), (1, g1_hbm)):
            out0_hbm = t_hbm.at[h]
            out1_hbm = t_hbm.at[2 + h]
            for t in range(RW // ZR):
                pltpu.async_copy(z_v, acc_sh.at[pl.ds(sid * RW + t * ZR, ZR)],
                                 gsems[0])
            for t in range(RW // ZR):
                pltpu.make_async_copy(
                    z_v, acc_sh.at[pl.ds(sid * RW + t * ZR, ZR)],
                    gsems[0]).wait()
            plsc.subcore_barrier()

            for b in range(NB):
                pltpu.async_copy(g_hbm.at[row_v.at[b]], gbufs[b], gsems[b])

            def group(gi, c):
                for b in range(NB):
                    j = gi * NB + b
                    pltpu.make_async_copy(
                        g_hbm.at[row_v.at[j]], gbufs[b], gsems[b]).wait()

                    @pl.when(gi > 0)
                    def _():
                        pltpu.make_async_copy(
                            sbufs[b], acc_sh.at[col_v.at[j]], ssems[b]).wait()

                    jbase = j * CH

                    def scale(gg, c2, _b=b, _jbase=jbase):
                        ew16 = ew_v[pl.ds(_jbase + gg * 16, 16)]
                        ebase = gg * 16
                        for e in range(16):
                            w_s = ew16[e]
                            for u in range(DH // 16):
                                sl = pl.ds(u * 16, 16)
                                sbufs[_b][ebase + e, sl] = \
                                    gbufs[_b][ebase + e, sl] * w_s
                        return c2

                    lax.fori_loop(0, CH // 16, scale, 0)

                    @pl.when(gi < NG - 1)
                    def _():
                        pltpu.async_copy(
                            g_hbm.at[row_v.at[j + NB]], gbufs[b], gsems[b])

                    pltpu.async_copy(
                        sbufs[b], acc_sh.at[col_v.at[j]], ssems[b], add=True)
                return c

            lax.fori_loop(0, NG, group, 0)
            for b in range(NB):
                pltpu.make_async_copy(
                    sbufs[b], acc_sh.at[col_v.at[K - NB + b]], ssems[b]).wait()
            plsc.subcore_barrier()
            WB = RW // CH
            for t in range(WB):
                p = t % 2
                if t >= 2:
                    @pl.when(cid == 0)
                    def _(_p=p, _t=t):
                        pltpu.make_async_copy(
                            sbufs[_p],
                            out0_hbm.at[pl.ds(base + (_t - 2) * CH, CH)],
                            ssems[_p]).wait()

                    @pl.when(cid == 1)
                    def _(_p=p, _t=t):
                        pltpu.make_async_copy(
                            sbufs[_p],
                            out1_hbm.at[pl.ds(base + (_t - 2) * CH, CH)],
                            ssems[_p]).wait()

                pltpu.async_copy(acc_sh.at[pl.ds(base + t * CH, CH)],
                                 sbufs[p], gsems[p])
                pltpu.make_async_copy(acc_sh.at[pl.ds(base + t * CH, CH)],
                                      sbufs[p], gsems[p]).wait()

                @pl.when(cid == 0)
                def _(_p=p, _t=t):
                    pltpu.async_copy(
                        sbufs[_p], out0_hbm.at[pl.ds(base + _t * CH, CH)],
                        ssems[_p])

                @pl.when(cid == 1)
                def _(_p=p, _t=t):
                    pltpu.async_copy(
                        sbufs[_p], out1_hbm.at[pl.ds(base + _t * CH, CH)],
                        ssems[_p])

            for t in range(WB - 2, WB):
                p = t % 2

                @pl.when(cid == 0)
                def _(_p=p, _t=t):
                    pltpu.make_async_copy(
                        sbufs[_p], out0_hbm.at[pl.ds(base + _t * CH, CH)],
                        ssems[_p]).wait()

                @pl.when(cid == 1)
                def _(_p=p, _t=t):
                    pltpu.make_async_copy(
                        sbufs[_p], out1_hbm.at[pl.ds(base + _t * CH, CH)],
                        ssems[_p]).wait()

            plsc.subcore_barrier()

    return prop_kernel(g0, g1, rowp, colp, ewp)


def _final_body(t_ref, g0_ref, g1_ref, dis_ref, b_ref, out_ref):
    r = dis_ref[...]
    b2 = b_ref[...]
    t = t_ref[...]
    out_ref[:, :DH] = (t[0] + t[2] + g0_ref[...]) * r + b2[:, :DH]
    out_ref[:, DH:] = (t[1] + t[3] + g1_ref[...]) * r + b2[:, DH:]


def _final_call(t_all, g0, g1, dis, b2):
    R = 2000
    grid = (N // R,)
    half = pl.BlockSpec((R, DH), lambda i: (i, 0))
    return pl.pallas_call(
        _final_body,
        grid=grid,
        in_specs=[
            pl.BlockSpec((4, R, DH), lambda i: (0, i, 0)),
            half, half,
            pl.BlockSpec((R, 1), lambda i: (i, 0)),
            pl.BlockSpec((1, D), lambda i: (0, 0)),
        ],
        out_specs=pl.BlockSpec((R, D), lambda i: (i, 0)),
        out_shape=jax.ShapeDtypeStruct((N, D), _f32),
    )(t_all, g0, g1, dis, b2)


def kernel(x, edge_index, edge_attr, W, b):
    E = edge_index.shape[1]
    eiflat = edge_index.astype(_i32).reshape(2 * E)
    row = eiflat[:E]
    col = eiflat[E:]
    ew = edge_attr.astype(_f32).reshape(E)
    pad = EP - E
    pidx = lax.iota(_i32, pad)
    rowp = jnp.concatenate([row, pidx % N]).reshape(EP // CH, CH)
    colp = jnp.concatenate([col, N + pidx % (NP - N)]).reshape(EP // CH, CH)
    ewp = jnp.concatenate([ew, jnp.zeros((pad,), _f32)])

    deg = _degree_call(colp, ewp)
    g0, g1, dis = _transform_call(x, W, deg)
    t_all = _propagate_call(g0, g1, rowp, colp, ewp)
    return _final_call(t_all, g0, g1, dis, b.reshape(1, D))


# speedup vs baseline: 1.0463x; 1.0463x over previous
import functools

import jax
import jax.numpy as jnp
from jax import lax
from jax.experimental import pallas as pl
from jax.experimental.pallas import tpu as pltpu
from jax.experimental.pallas import tpu_sc as plsc

_f32 = jnp.float32
_i32 = jnp.int32

N = 10000
D = 128
NC, NS = 2, 16
NW = NC * NS
CH = 128
K = 81
EPW = K * CH
EP = NW * EPW
NP = 10240
RW = NP // NS
ZR = 64


def _mesh():
    return plsc.VectorSubcoreMesh(core_axis_name="c", subcore_axis_name="s")


def _degree_call(colp, ewp):
    sds = jax.ShapeDtypeStruct((2, N), _f32)

    @functools.partial(
        pl.kernel,
        out_type=sds,
        mesh=_mesh(),
        scratch_types=[
            pltpu.VMEM((K, CH), _i32),
            pltpu.VMEM((K, CH), _f32),
            pltpu.VMEM((RW,), _f32),
            pltpu.VMEM_SHARED((NP,), _f32),
        ],
        compiler_params=pltpu.CompilerParams(use_tc_tiling_on_sc=False),
    )
    def deg_kernel(col_hbm, ew_hbm, deg_hbm, col_v, ew_v, z_v, deg_sh):
        cid = lax.axis_index("c")
        sid = lax.axis_index("s")
        wid = sid * NC + cid

        def zi(i, c):
            z_v[pl.ds(i * 16, 16)] = jnp.zeros((16,), _f32)
            return c

        lax.fori_loop(0, RW // 16, zi, 0)
        pltpu.sync_copy(z_v, deg_sh.at[pl.ds(sid * RW, RW)])
        pltpu.sync_copy(col_hbm.at[pl.ds(wid * K, K)], col_v)
        pltpu.sync_copy(ew_hbm.at[pl.ds(wid * K, K)], ew_v)
        plsc.subcore_barrier()

        def chunk(j, c):
            pltpu.sync_copy(ew_v.at[j], deg_sh.at[col_v.at[j]], add=True)
            return c

        lax.fori_loop(0, K, chunk, 0)
        plsc.subcore_barrier()
        base = jnp.minimum(sid * RW, N - RW)
        pltpu.sync_copy(deg_sh.at[pl.ds(base, RW)], z_v)

        @pl.when(cid == 0)
        def _():
            pltpu.sync_copy(z_v, deg_hbm.at[0, pl.ds(base, RW)])

        @pl.when(cid == 1)
        def _():
            pltpu.sync_copy(z_v, deg_hbm.at[1, pl.ds(base, RW)])

    return deg_kernel(colp, ewp)


DH = D // 2


def _transform_body(x_ref, w_ref, d_ref, g0_ref, g1_ref, dis_ref):
    h = lax.dot_general(
        x_ref[...], w_ref[...], (((1,), (1,)), ((), ())),
        preferred_element_type=_f32,
    )
    ones2 = jnp.ones((2, 1), _f32)
    dcol = lax.dot_general(d_ref[...], ones2, (((0,), (0,)), ((), ())),
                           precision=lax.Precision.HIGHEST,
                           preferred_element_type=_f32)
    r = lax.rsqrt(dcol + 1.0)
    g = h * r
    g0_ref[...] = g[:, :DH]
    g1_ref[...] = g[:, DH:]
    dis_ref[...] = r


def _transform_call(x, w, deg):
    return pl.pallas_call(
        _transform_body,
        grid=(1,),
        in_specs=[
            pl.BlockSpec((N, D), lambda i: (0, 0)),
            pl.BlockSpec((D, D), lambda i: (0, 0)),
            pl.BlockSpec((2, N), lambda i: (0, 0)),
        ],
        out_specs=[
            pl.BlockSpec((N, DH), lambda i: (0, 0)),
            pl.BlockSpec((N, DH), lambda i: (0, 0)),
            pl.BlockSpec((N, 1), lambda i: (0, 0)),
        ],
        out_shape=[
            jax.ShapeDtypeStruct((N, DH), _f32),
            jax.ShapeDtypeStruct((N, DH), _f32),
            jax.ShapeDtypeStruct((N, 1), _f32),
        ],
    )(x, w, deg)


def _propagate_call(g0, g1, rowp, colp, ewp):
    sds = jax.ShapeDtypeStruct((N, DH), _f32)

    NB = 3

    @functools.partial(
        pl.kernel,
        out_type=jax.ShapeDtypeStruct((4, N, DH), _f32),
        mesh=_mesh(),
        scratch_types=[
            pltpu.VMEM((K, CH), _i32),
            pltpu.VMEM((K, CH), _i32),
            pltpu.VMEM((K, CH), _f32),
            [pltpu.VMEM((CH, DH), _f32) for _ in range(NB)],
            [pltpu.VMEM((CH, DH), _f32) for _ in range(NB)],
            pltpu.VMEM((ZR, DH), _f32),
            pltpu.VMEM_SHARED((NP, DH), _f32),
            [pltpu.SemaphoreType.DMA for _ in range(NB)],
            [pltpu.SemaphoreType.DMA for _ in range(NB)],
        ],
        compiler_params=pltpu.CompilerParams(use_tc_tiling_on_sc=False),
    )
    def prop_kernel(g0_hbm, g1_hbm, row_hbm, col_hbm, ew_hbm, t_hbm,
                    row_v, col_v, ew_v, gbufs, sbufs, z_v, acc_sh,
                    gsems, ssems):
        cid = lax.axis_index("c")
        sid = lax.axis_index("s")
        wid = sid * NC + cid

        def zi(i, c):
            for u in range(DH // 16):
                z_v[i, pl.ds(u * 16, 16)] = jnp.zeros((16,), _f32)
            return c

        lax.fori_loop(0, ZR, zi, 0)
        pltpu.sync_copy(row_hbm.at[pl.ds(wid * K, K)], row_v)
        pltpu.sync_copy(col_hbm.at[pl.ds(wid * K, K)], col_v)
        pltpu.sync_copy(ew_hbm.at[pl.ds(wid * K, K)], ew_v)
        base = jnp.minimum(sid * RW, N - RW)
        NG = K // NB

        for h, g_hbm in ((0, g0_hbm), (1, g1_hbm)):
            out0_hbm = t_hbm.at[h]
            out1_hbm = t_hbm.at[2 + h]
            for t in range(RW // ZR):
                pltpu.async_copy(z_v, acc_sh.at[pl.ds(sid * RW + t * ZR, ZR)],
                                 gsems[0])
            for t in range(RW // ZR):
                pltpu.make_async_copy(
                    z_v, acc_sh.at[pl.ds(sid * RW + t * ZR, ZR)],
                    gsems[0]).wait()
            plsc.subcore_barrier()

            for b in range(NB):
                pltpu.async_copy(g_hbm.at[row_v.at[b]], gbufs[b], gsems[b])

            def group(gi, c):
                for b in range(NB):
                    j = gi * NB + b
                    pltpu.make_async_copy(
                        g_hbm.at[row_v.at[j]], gbufs[b], gsems[b]).wait()

                    @pl.when(gi > 0)
                    def _():
                        pltpu.make_async_copy(
                            sbufs[b], acc_sh.at[col_v.at[j]], ssems[b]).wait()

                    def scale(gg, c2, _b=b, _j=j):
                        ew16 = ew_v[_j, pl.ds(gg * 16, 16)]
                        ebase = gg * 16
                        for e in range(16):
                            w_s = ew16[e]
                            for u in range(DH // 16):
                                sl = pl.ds(u * 16, 16)
                                sbufs[_b][ebase + e, sl] = \
                                    gbufs[_b][ebase + e, sl] * w_s
                        return c2

                    lax.fori_loop(0, CH // 16, scale, 0)

                    @pl.when(gi < NG - 1)
                    def _():
                        pltpu.async_copy(
                            g_hbm.at[row_v.at[j + NB]], gbufs[b], gsems[b])

                    pltpu.async_copy(
                        sbufs[b], acc_sh.at[col_v.at[j]], ssems[b], add=True)
                return c

            lax.fori_loop(0, NG, group, 0)
            for b in range(NB):
                pltpu.make_async_copy(
                    sbufs[b], acc_sh.at[col_v.at[K - NB + b]], ssems[b]).wait()
            plsc.subcore_barrier()
            WB = RW // CH
            for t in range(WB):
                p = t % 2
                if t >= 2:
                    @pl.when(cid == 0)
                    def _(_p=p, _t=t):
                        pltpu.make_async_copy(
                            sbufs[_p],
                            out0_hbm.at[pl.ds(base + (_t - 2) * CH, CH)],
                            ssems[_p]).wait()

                    @pl.when(cid == 1)
                    def _(_p=p, _t=t):
                        pltpu.make_async_copy(
                            sbufs[_p],
                            out1_hbm.at[pl.ds(base + (_t - 2) * CH, CH)],
                            ssems[_p]).wait()

                pltpu.async_copy(acc_sh.at[pl.ds(base + t * CH, CH)],
                                 sbufs[p], gsems[p])
                pltpu.make_async_copy(acc_sh.at[pl.ds(base + t * CH, CH)],
                                      sbufs[p], gsems[p]).wait()

                @pl.when(cid == 0)
                def _(_p=p, _t=t):
                    pltpu.async_copy(
                        sbufs[_p], out0_hbm.at[pl.ds(base + _t * CH, CH)],
                        ssems[_p])

                @pl.when(cid == 1)
                def _(_p=p, _t=t):
                    pltpu.async_copy(
                        sbufs[_p], out1_hbm.at[pl.ds(base + _t * CH, CH)],
                        ssems[_p])

            for t in range(WB - 2, WB):
                p = t % 2

                @pl.when(cid == 0)
                def _(_p=p, _t=t):
                    pltpu.make_async_copy(
                        sbufs[_p], out0_hbm.at[pl.ds(base + _t * CH, CH)],
                        ssems[_p]).wait()

                @pl.when(cid == 1)
                def _(_p=p, _t=t):
                    pltpu.make_async_copy(
                        sbufs[_p], out1_hbm.at[pl.ds(base + _t * CH, CH)],
                        ssems[_p]).wait()

            plsc.subcore_barrier()

    return prop_kernel(g0, g1, rowp, colp, ewp)


def _final_body(t_ref, g0_ref, g1_ref, dis_ref, b_ref, out_ref):
    r = dis_ref[...]
    b2 = b_ref[...]
    t = t_ref[...]
    out_ref[:, :DH] = (t[0] + t[2] + g0_ref[...]) * r + b2[:, :DH]
    out_ref[:, DH:] = (t[1] + t[3] + g1_ref[...]) * r + b2[:, DH:]


def _final_call(t_all, g0, g1, dis, b2):
    R = 2000
    grid = (N // R,)
    half = pl.BlockSpec((R, DH), lambda i: (i, 0))
    return pl.pallas_call(
        _final_body,
        grid=grid,
        in_specs=[
            pl.BlockSpec((4, R, DH), lambda i: (0, i, 0)),
            half, half,
            pl.BlockSpec((R, 1), lambda i: (i, 0)),
            pl.BlockSpec((1, D), lambda i: (0, 0)),
        ],
        out_specs=pl.BlockSpec((R, D), lambda i: (i, 0)),
        out_shape=jax.ShapeDtypeStruct((N, D), _f32),
    )(t_all, g0, g1, dis, b2)


def kernel(x, edge_index, edge_attr, W, b):
    E = edge_index.shape[1]
    eiflat = edge_index.astype(_i32).reshape(2 * E)
    row = eiflat[:E]
    col = eiflat[E:]
    pad = EP - E
    pidx = lax.iota(_i32, pad)
    rowp = jnp.concatenate([row, pidx % N]).reshape(EP // CH, CH)
    colp = jnp.concatenate([col, N + pidx % (NP - N)]).reshape(EP // CH, CH)
    ewp = jnp.concatenate(
        [edge_attr.astype(_f32), jnp.zeros((pad, 1), _f32)]
    ).reshape(EP // CH, CH)

    deg = _degree_call(colp, ewp)
    g0, g1, dis = _transform_call(x, W, deg)
    t_all = _propagate_call(g0, g1, rowp, colp, ewp)
    return _final_call(t_all, g0, g1, dis, b.reshape(1, D))


# speedup vs baseline: 1.0554x; 1.0087x over previous
import functools

import jax
import jax.numpy as jnp
from jax import lax
from jax.experimental import pallas as pl
from jax.experimental.pallas import tpu as pltpu
from jax.experimental.pallas import tpu_sc as plsc

_f32 = jnp.float32
_i32 = jnp.int32

N = 10000
D = 128
NC, NS = 2, 16
NW = NC * NS
CH = 128
K = 81
EPW = K * CH
EP = NW * EPW
NP = 10240
RW = NP // NS
ZR = 64


def _mesh():
    return plsc.VectorSubcoreMesh(core_axis_name="c", subcore_axis_name="s")


def _degree_call(rcp, ewp):
    sds = jax.ShapeDtypeStruct((2, N), _f32)

    @functools.partial(
        pl.kernel,
        out_type=sds,
        mesh=_mesh(),
        scratch_types=[
            pltpu.VMEM((K, CH), _i32),
            pltpu.VMEM((K, CH), _f32),
            pltpu.VMEM((RW,), _f32),
            pltpu.VMEM_SHARED((NP,), _f32),
        ],
        compiler_params=pltpu.CompilerParams(use_tc_tiling_on_sc=False),
    )
    def deg_kernel(rc_hbm, ew_hbm, deg_hbm, col_v, ew_v, z_v, deg_sh):
        cid = lax.axis_index("c")
        sid = lax.axis_index("s")
        wid = sid * NC + cid

        def zi(i, c):
            z_v[pl.ds(i * 16, 16)] = jnp.zeros((16,), _f32)
            return c

        lax.fori_loop(0, RW // 16, zi, 0)
        pltpu.sync_copy(z_v, deg_sh.at[pl.ds(sid * RW, RW)])
        pltpu.sync_copy(rc_hbm.at[1, pl.ds(wid * K, K)], col_v)
        pltpu.sync_copy(ew_hbm.at[pl.ds(wid * K, K)], ew_v)
        plsc.subcore_barrier()

        def chunk(j, c):
            pltpu.sync_copy(ew_v.at[j], deg_sh.at[col_v.at[j]], add=True)
            return c

        lax.fori_loop(0, K, chunk, 0)
        plsc.subcore_barrier()
        base = jnp.minimum(sid * RW, N - RW)
        pltpu.sync_copy(deg_sh.at[pl.ds(base, RW)], z_v)

        @pl.when(cid == 0)
        def _():
            pltpu.sync_copy(z_v, deg_hbm.at[0, pl.ds(base, RW)])

        @pl.when(cid == 1)
        def _():
            pltpu.sync_copy(z_v, deg_hbm.at[1, pl.ds(base, RW)])

    return deg_kernel(rcp, ewp)


DH = D // 2


def _transform_body(x_ref, w_ref, d_ref, g0_ref, g1_ref, dis_ref):
    h = lax.dot_general(
        x_ref[...], w_ref[...], (((1,), (1,)), ((), ())),
        preferred_element_type=_f32,
    )
    ones2 = jnp.ones((2, 1), _f32)
    dcol = lax.dot_general(d_ref[...], ones2, (((0,), (0,)), ((), ())),
                           precision=lax.Precision.HIGHEST,
                           preferred_element_type=_f32)
    r = lax.rsqrt(dcol + 1.0)
    g = h * r
    g0_ref[...] = g[:, :DH]
    g1_ref[...] = g[:, DH:]
    dis_ref[...] = r


def _transform_call(x, w, deg):
    return pl.pallas_call(
        _transform_body,
        grid=(1,),
        in_specs=[
            pl.BlockSpec((N, D), lambda i: (0, 0)),
            pl.BlockSpec((D, D), lambda i: (0, 0)),
            pl.BlockSpec((2, N), lambda i: (0, 0)),
        ],
        out_specs=[
            pl.BlockSpec((N, DH), lambda i: (0, 0)),
            pl.BlockSpec((N, DH), lambda i: (0, 0)),
            pl.BlockSpec((N, 1), lambda i: (0, 0)),
        ],
        out_shape=[
            jax.ShapeDtypeStruct((N, DH), _f32),
            jax.ShapeDtypeStruct((N, DH), _f32),
            jax.ShapeDtypeStruct((N, 1), _f32),
        ],
    )(x, w, deg)


def _propagate_call(g0, g1, rcp, ewp):
    sds = jax.ShapeDtypeStruct((N, DH), _f32)

    NB = 3

    @functools.partial(
        pl.kernel,
        out_type=jax.ShapeDtypeStruct((4, N, DH), _f32),
        mesh=_mesh(),
        scratch_types=[
            pltpu.VMEM((K, CH), _i32),
            pltpu.VMEM((K, CH), _i32),
            pltpu.VMEM((K, CH), _f32),
            [pltpu.VMEM((CH, DH), _f32) for _ in range(NB)],
            [pltpu.VMEM((CH, DH), _f32) for _ in range(NB)],
            pltpu.VMEM((ZR, DH), _f32),
            pltpu.VMEM_SHARED((NP, DH), _f32),
            [pltpu.SemaphoreType.DMA for _ in range(NB)],
            [pltpu.SemaphoreType.DMA for _ in range(NB)],
        ],
        compiler_params=pltpu.CompilerParams(use_tc_tiling_on_sc=False),
    )
    def prop_kernel(g0_hbm, g1_hbm, rc_hbm, ew_hbm, t_hbm,
                    row_v, col_v, ew_v, gbufs, sbufs, z_v, acc_sh,
                    gsems, ssems):
        cid = lax.axis_index("c")
        sid = lax.axis_index("s")
        wid = sid * NC + cid

        def zi(i, c):
            for u in range(DH // 16):
                z_v[i, pl.ds(u * 16, 16)] = jnp.zeros((16,), _f32)
            return c

        lax.fori_loop(0, ZR, zi, 0)
        pltpu.sync_copy(rc_hbm.at[0, pl.ds(wid * K, K)], row_v)
        pltpu.sync_copy(rc_hbm.at[1, pl.ds(wid * K, K)], col_v)
        pltpu.sync_copy(ew_hbm.at[pl.ds(wid * K, K)], ew_v)
        base = jnp.minimum(sid * RW, N - RW)
        NG = K // NB

        for h, g_hbm in ((0, g0_hbm), (1, g1_hbm)):
            out0_hbm = t_hbm.at[h]
            out1_hbm = t_hbm.at[2 + h]
            for t in range(RW // ZR):
                pltpu.async_copy(z_v, acc_sh.at[pl.ds(sid * RW + t * ZR, ZR)],
                                 gsems[0])
            for t in range(RW // ZR):
                pltpu.make_async_copy(
                    z_v, acc_sh.at[pl.ds(sid * RW + t * ZR, ZR)],
                    gsems[0]).wait()
            plsc.subcore_barrier()

            for b in range(NB):
                pltpu.async_copy(g_hbm.at[row_v.at[b]], gbufs[b], gsems[b])

            def group(gi, c):
                for b in range(NB):
                    j = gi * NB + b
                    pltpu.make_async_copy(
                        g_hbm.at[row_v.at[j]], gbufs[b], gsems[b]).wait()

                    @pl.when(gi > 0)
                    def _():
                        pltpu.make_async_copy(
                            sbufs[b], acc_sh.at[col_v.at[j]], ssems[b]).wait()

                    def scale(gg, c2, _b=b, _j=j):
                        ew16 = ew_v[_j, pl.ds(gg * 16, 16)]
                        ebase = gg * 16
                        for e in range(16):
                            w_s = ew16[e]
                            for u in range(DH // 16):
                                sl = pl.ds(u * 16, 16)
                                sbufs[_b][ebase + e, sl] = \
                                    gbufs[_b][ebase + e, sl] * w_s
                        return c2

                    lax.fori_loop(0, CH // 16, scale, 0)

                    @pl.when(gi < NG - 1)
                    def _():
                        pltpu.async_copy(
                            g_hbm.at[row_v.at[j + NB]], gbufs[b], gsems[b])

                    pltpu.async_copy(
                        sbufs[b], acc_sh.at[col_v.at[j]], ssems[b], add=True)
                return c

            lax.fori_loop(0, NG, group, 0)
            for b in range(NB):
                pltpu.make_async_copy(
                    sbufs[b], acc_sh.at[col_v.at[K - NB + b]], ssems[b]).wait()
            plsc.subcore_barrier()
            WB = RW // CH
            for t in range(WB):
                p = t % 2
                if t >= 2:
                    @pl.when(cid == 0)
                    def _(_p=p, _t=t):
                        pltpu.make_async_copy(
                            sbufs[_p],
                            out0_hbm.at[pl.ds(base + (_t - 2) * CH, CH)],
                            ssems[_p]).wait()

                    @pl.when(cid == 1)
                    def _(_p=p, _t=t):
                        pltpu.make_async_copy(
                            sbufs[_p],
                            out1_hbm.at[pl.ds(base + (_t - 2) * CH, CH)],
                            ssems[_p]).wait()

                pltpu.async_copy(acc_sh.at[pl.ds(base + t * CH, CH)],
                                 sbufs[p], gsems[p])
                pltpu.make_async_copy(acc_sh.at[pl.ds(base + t * CH, CH)],
                                      sbufs[p], gsems[p]).wait()

                @pl.when(cid == 0)
                def _(_p=p, _t=t):
                    pltpu.async_copy(
                        sbufs[_p], out0_hbm.at[pl.ds(base + _t * CH, CH)],
                        ssems[_p])

                @pl.when(cid == 1)
                def _(_p=p, _t=t):
                    pltpu.async_copy(
                        sbufs[_p], out1_hbm.at[pl.ds(base + _t * CH, CH)],
                        ssems[_p])

            for t in range(WB - 2, WB):
                p = t % 2

                @pl.when(cid == 0)
                def _(_p=p, _t=t):
                    pltpu.make_async_copy(
                        sbufs[_p], out0_hbm.at[pl.ds(base + _t * CH, CH)],
                        ssems[_p]).wait()

                @pl.when(cid == 1)
                def _(_p=p, _t=t):
                    pltpu.make_async_copy(
                        sbufs[_p], out1_hbm.at[pl.ds(base + _t * CH, CH)],
                        ssems[_p]).wait()

            plsc.subcore_barrier()

    return prop_kernel(g0, g1, rcp, ewp)


def _final_body(t_ref, g0_ref, g1_ref, dis_ref, b_ref, out_ref):
    r = dis_ref[...]
    b2 = b_ref[...]
    t = t_ref[...]
    out_ref[:, :DH] = (t[0] + t[2] + g0_ref[...]) * r + b2[:, :DH]
    out_ref[:, DH:] = (t[1] + t[3] + g1_ref[...]) * r + b2[:, DH:]


def _final_call(t_all, g0, g1, dis, b2):
    R = 2000
    grid = (N // R,)
    half = pl.BlockSpec((R, DH), lambda i: (i, 0))
    return pl.pallas_call(
        _final_body,
        grid=grid,
        in_specs=[
            pl.BlockSpec((4, R, DH), lambda i: (0, i, 0)),
            half, half,
            pl.BlockSpec((R, 1), lambda i: (i, 0)),
            pl.BlockSpec((1, D), lambda i: (0, 0)),
        ],
        out_specs=pl.BlockSpec((R, D), lambda i: (i, 0)),
        out_shape=jax.ShapeDtypeStruct((N, D), _f32),
    )(t_all, g0, g1, dis, b2)


def kernel(x, edge_index, edge_attr, W, b):
    E = edge_index.shape[1]
    pad = EP - E
    pidx = lax.iota(_i32, pad)
    eipad = jnp.stack([pidx % N, N + pidx % (NP - N)])
    rcp = jnp.concatenate([edge_index.astype(_i32), eipad],
                          axis=1).reshape(2, EP // CH, CH)
    ewp = jnp.concatenate(
        [edge_attr.astype(_f32), jnp.zeros((pad, 1), _f32)]
    ).reshape(EP // CH, CH)

    deg = _degree_call(rcp, ewp)
    g0, g1, dis = _transform_call(x, W, deg)
    t_all = _propagate_call(g0, g1, rcp, ewp)
    return _final_call(t_all, g0, g1, dis, b.reshape(1, D))


# speedup vs baseline: 1.0554x; 1.0000x over previous
import functools

import jax
import jax.numpy as jnp
from jax import lax
from jax.experimental import pallas as pl
from jax.experimental.pallas import tpu as pltpu
from jax.experimental.pallas import tpu_sc as plsc

_f32 = jnp.float32
_i32 = jnp.int32

N = 10000
D = 128
NC, NS = 2, 16
NW = NC * NS
CH = 128
K = 81
EPW = K * CH
EP = NW * EPW
NP = 10240
RW = NP // NS
ZR = 64


def _mesh():
    return plsc.VectorSubcoreMesh(core_axis_name="c", subcore_axis_name="s")


def _degree_call(rcp, ewp):
    sds = jax.ShapeDtypeStruct((2, N), _f32)

    @functools.partial(
        pl.kernel,
        out_type=sds,
        mesh=_mesh(),
        scratch_types=[
            pltpu.VMEM((K, CH), _i32),
            pltpu.VMEM((K, CH), _f32),
            pltpu.VMEM((RW,), _f32),
            pltpu.VMEM_SHARED((NP,), _f32),
        ],
        compiler_params=pltpu.CompilerParams(use_tc_tiling_on_sc=False),
    )
    def deg_kernel(rc_hbm, ew_hbm, deg_hbm, col_v, ew_v, z_v, deg_sh):
        cid = lax.axis_index("c")
        sid = lax.axis_index("s")
        wid = sid * NC + cid

        def zi(i, c):
            z_v[pl.ds(i * 16, 16)] = jnp.zeros((16,), _f32)
            return c

        lax.fori_loop(0, RW // 16, zi, 0)
        pltpu.sync_copy(z_v, deg_sh.at[pl.ds(sid * RW, RW)])
        pltpu.sync_copy(rc_hbm.at[1, pl.ds(wid * K, K)], col_v)
        pltpu.sync_copy(ew_hbm.at[pl.ds(wid * K, K)], ew_v)
        plsc.subcore_barrier()

        def chunk(j, c):
            pltpu.sync_copy(ew_v.at[j], deg_sh.at[col_v.at[j]], add=True)
            return c

        lax.fori_loop(0, K, chunk, 0)
        plsc.subcore_barrier()
        base = jnp.minimum(sid * RW, N - RW)
        pltpu.sync_copy(deg_sh.at[pl.ds(base, RW)], z_v)

        @pl.when(cid == 0)
        def _():
            pltpu.sync_copy(z_v, deg_hbm.at[0, pl.ds(base, RW)])

        @pl.when(cid == 1)
        def _():
            pltpu.sync_copy(z_v, deg_hbm.at[1, pl.ds(base, RW)])

    return deg_kernel(rcp, ewp)


DH = D // 2


def _transform_body(x_ref, w_ref, d_ref, g0_ref, g1_ref, dis_ref):
    h = lax.dot_general(
        x_ref[...], w_ref[...], (((1,), (1,)), ((), ())),
        preferred_element_type=_f32,
    )
    ones2 = jnp.ones((2, 1), _f32)
    dcol = lax.dot_general(d_ref[...], ones2, (((0,), (0,)), ((), ())),
                           precision=lax.Precision.HIGHEST,
                           preferred_element_type=_f32)
    r = lax.rsqrt(dcol + 1.0)
    g = h * r
    g0_ref[...] = g[:, :DH]
    g1_ref[...] = g[:, DH:]
    dis_ref[...] = r


def _transform_call(x, w, deg):
    return pl.pallas_call(
        _transform_body,
        grid=(1,),
        in_specs=[
            pl.BlockSpec((N, D), lambda i: (0, 0)),
            pl.BlockSpec((D, D), lambda i: (0, 0)),
            pl.BlockSpec((2, N), lambda i: (0, 0)),
        ],
        out_specs=[
            pl.BlockSpec((N, DH), lambda i: (0, 0)),
            pl.BlockSpec((N, DH), lambda i: (0, 0)),
            pl.BlockSpec((N, 1), lambda i: (0, 0)),
        ],
        out_shape=[
            jax.ShapeDtypeStruct((N, DH), _f32),
            jax.ShapeDtypeStruct((N, DH), _f32),
            jax.ShapeDtypeStruct((N, 1), _f32),
        ],
    )(x, w, deg)


def _propagate_call(g0, g1, rcp, ewp):
    NB = 3

    @functools.partial(
        pl.kernel,
        out_type=jax.ShapeDtypeStruct((4, N, DH), _f32),
        mesh=_mesh(),
        scratch_types=[
            pltpu.VMEM((K, CH), _i32),
            pltpu.VMEM((K, CH), _i32),
            pltpu.VMEM((K, CH), _f32),
            [pltpu.VMEM((CH, DH), _f32) for _ in range(NB)],
            [pltpu.VMEM((CH, DH), _f32) for _ in range(NB)],
            pltpu.VMEM((ZR, DH), _f32),
            pltpu.VMEM_SHARED((NP, DH), _f32),
            [pltpu.SemaphoreType.DMA for _ in range(NB)],
            [pltpu.SemaphoreType.DMA for _ in range(NB)],
        ],
        compiler_params=pltpu.CompilerParams(use_tc_tiling_on_sc=False),
    )
    def prop_kernel(g0_hbm, g1_hbm, rc_hbm, ew_hbm, t_hbm,
                    row_v, col_v, ew_v, gbufs, sbufs, z_v, acc_sh,
                    gsems, ssems):
        cid = lax.axis_index("c")
        sid = lax.axis_index("s")
        wid = sid * NC + cid

        def zi(i, c):
            for u in range(DH // 16):
                z_v[i, pl.ds(u * 16, 16)] = jnp.zeros((16,), _f32)
            return c

        lax.fori_loop(0, ZR, zi, 0)
        pltpu.sync_copy(rc_hbm.at[0, pl.ds(wid * K, K)], row_v)
        pltpu.sync_copy(rc_hbm.at[1, pl.ds(wid * K, K)], col_v)
        pltpu.sync_copy(ew_hbm.at[pl.ds(wid * K, K)], ew_v)
        base = jnp.minimum(sid * RW, N - RW)
        NG = K // NB

        for h, g_hbm in ((0, g0_hbm), (1, g1_hbm)):
            out0_hbm = t_hbm.at[h]
            out1_hbm = t_hbm.at[2 + h]
            for t in range(RW // ZR):
                pltpu.async_copy(z_v, acc_sh.at[pl.ds(sid * RW + t * ZR, ZR)],
                                 gsems[0])
            for t in range(RW // ZR):
                pltpu.make_async_copy(
                    z_v, acc_sh.at[pl.ds(sid * RW + t * ZR, ZR)],
                    gsems[0]).wait()
            plsc.subcore_barrier()

            for b in range(NB):
                pltpu.async_copy(g_hbm.at[row_v.at[b]], gbufs[b], gsems[b])

            def group(gi, c):
                for b in range(NB):
                    j = gi * NB + b
                    pltpu.make_async_copy(
                        g_hbm.at[row_v.at[j]], gbufs[b], gsems[b]).wait()

                    @pl.when(gi > 0)
                    def _():
                        pltpu.make_async_copy(
                            sbufs[b], acc_sh.at[col_v.at[j]], ssems[b]).wait()

                    def scale(gg, c2, _b=b, _j=j):
                        ew16 = ew_v[_j, pl.ds(gg * 16, 16)]
                        ebase = gg * 16
                        for e in range(16):
                            w_s = ew16[e]
                            for u in range(DH // 16):
                                sl = pl.ds(u * 16, 16)
                                sbufs[_b][ebase + e, sl] = \
                                    gbufs[_b][ebase + e, sl] * w_s
                        return c2

                    lax.fori_loop(0, CH // 16, scale, 0)

                    @pl.when(gi < NG - 1)
                    def _():
                        pltpu.async_copy(
                            g_hbm.at[row_v.at[j + NB]], gbufs[b], gsems[b])

                    pltpu.async_copy(
                        sbufs[b], acc_sh.at[col_v.at[j]], ssems[b], add=True)
                return c

            lax.fori_loop(0, NG, group, 0)
            for b in range(NB):
                pltpu.make_async_copy(
                    sbufs[b], acc_sh.at[col_v.at[K - NB + b]], ssems[b]).wait()
            plsc.subcore_barrier()
            WB = RW // CH
            for t in range(WB):
                p = t % 2
                if t >= 2:
                    @pl.when(cid == 0)
                    def _(_p=p, _t=t):
                        pltpu.make_async_copy(
                            sbufs[_p],
                            out0_hbm.at[pl.ds(base + (_t - 2) * CH, CH)],
                            ssems[_p]).wait()

                    @pl.when(cid == 1)
                    def _(_p=p, _t=t):
                        pltpu.make_async_copy(
                            sbufs[_p],
                            out1_hbm.at[pl.ds(base + (_t - 2) * CH, CH)],
                            ssems[_p]).wait()

                pltpu.async_copy(acc_sh.at[pl.ds(base + t * CH, CH)],
                                 sbufs[p], gsems[p])
                pltpu.make_async_copy(acc_sh.at[pl.ds(base + t * CH, CH)],
                                      sbufs[p], gsems[p]).wait()

                @pl.when(cid == 0)
                def _(_p=p, _t=t):
                    pltpu.async_copy(
                        sbufs[_p], out0_hbm.at[pl.ds(base + _t * CH, CH)],
                        ssems[_p])

                @pl.when(cid == 1)
                def _(_p=p, _t=t):
                    pltpu.async_copy(
                        sbufs[_p], out1_hbm.at[pl.ds(base + _t * CH, CH)],
                        ssems[_p])

            for t in range(WB - 2, WB):
                p = t % 2

                @pl.when(cid == 0)
                def _(_p=p, _t=t):
                    pltpu.make_async_copy(
                        sbufs[_p], out0_hbm.at[pl.ds(base + _t * CH, CH)],
                        ssems[_p]).wait()

                @pl.when(cid == 1)
                def _(_p=p, _t=t):
                    pltpu.make_async_copy(
                        sbufs[_p], out1_hbm.at[pl.ds(base + _t * CH, CH)],
                        ssems[_p]).wait()

            plsc.subcore_barrier()

    return prop_kernel(g0, g1, rcp, ewp)


def _final_body(t_ref, g0_ref, g1_ref, dis_ref, b_ref, out_ref):
    r = dis_ref[...]
    b2 = b_ref[...]
    t = t_ref[...]
    out_ref[:, :DH] = (t[0] + t[2] + g0_ref[...]) * r + b2[:, :DH]
    out_ref[:, DH:] = (t[1] + t[3] + g1_ref[...]) * r + b2[:, DH:]


def _final_call(t_all, g0, g1, dis, b2):
    R = 2000
    grid = (N // R,)
    half = pl.BlockSpec((R, DH), lambda i: (i, 0))
    return pl.pallas_call(
        _final_body,
        grid=grid,
        in_specs=[
            pl.BlockSpec((4, R, DH), lambda i: (0, i, 0)),
            half, half,
            pl.BlockSpec((R, 1), lambda i: (i, 0)),
            pl.BlockSpec((1, D), lambda i: (0, 0)),
        ],
        out_specs=pl.BlockSpec((R, D), lambda i: (i, 0)),
        out_shape=jax.ShapeDtypeStruct((N, D), _f32),
    )(t_all, g0, g1, dis, b2)


def kernel(x, edge_index, edge_attr, W, b):
    E = edge_index.shape[1]
    pad = EP - E
    pidx = lax.iota(_i32, pad)
    eipad = jnp.stack([pidx % N, N + pidx % (NP - N)])
    rcp = jnp.concatenate([edge_index.astype(_i32), eipad],
                          axis=1).reshape(2, EP // CH, CH)
    ewp = jnp.concatenate(
        [edge_attr.astype(_f32), jnp.zeros((pad, 1), _f32)]
    ).reshape(EP // CH, CH)

    deg = _degree_call(rcp, ewp)
    g0, g1, dis = _transform_call(x, W, deg)
    t_all = _propagate_call(g0, g1, rcp, ewp)
    return _final_call(t_all, g0, g1, dis, b.reshape(1, D))
